# baseline (speedup 1.0000x reference)
import functools
import jax
import jax.numpy as jnp
from jax import lax
from jax.experimental import pallas as pl
from jax.experimental.pallas import tpu as pltpu
from jax.experimental.pallas import tpu_sc as plsc

B = 16384
D = 32
L = 16
NC = 2
NS = 16
NW = NC * NS
BPW = B // NW
CHUNK = 128
NCH = BPW // CHUNK


def _nfm_body(uidx_hbm, iidx_hbm, utab_hbm, itab_hbm,
              w0_hbm, b0_hbm, w1_hbm, b1_hbm, out_hbm,
              uidx_v, iidx_v, urows_v, irows_v,
              w0_v, w1_v, b0_v, b1_v, out_v,
              sem_u, sem_i):
    cid = lax.axis_index("c")
    sid = lax.axis_index("s")
    wid = sid * NC + cid
    base = wid * BPW

    pltpu.sync_copy(uidx_hbm.at[pl.ds(base, BPW)], uidx_v)
    pltpu.sync_copy(iidx_hbm.at[pl.ds(base, BPW)], iidx_v)

    copies = []
    for k in range(NCH):
        copies.append(pltpu.async_copy(
            utab_hbm.at[uidx_v.at[pl.ds(k * CHUNK, CHUNK)]],
            urows_v.at[pl.ds(k * CHUNK, CHUNK), :], sem_u))
        copies.append(pltpu.async_copy(
            itab_hbm.at[iidx_v.at[pl.ds(k * CHUNK, CHUNK)]],
            irows_v.at[pl.ds(k * CHUNK, CHUNK), :], sem_i))

    pltpu.sync_copy(w0_hbm, w0_v)
    pltpu.sync_copy(w1_hbm, w1_v)
    pltpu.sync_copy(b0_hbm, b0_v)
    pltpu.sync_copy(b1_hbm, b1_v)

    s0 = jnp.zeros((L,), jnp.float32)
    s1 = jnp.zeros((L,), jnp.float32)
    for j in range(D):
        s0 = s0 + w1_v[j, pl.ds(0, L)]
        s1 = s1 + w1_v[j, pl.ds(L, L)]

    v0 = jnp.zeros((L,), jnp.float32)
    v1 = jnp.zeros((L,), jnp.float32)
    for j in range(D):
        sj = s0[j] if j < L else s1[j - L]
        v0 = v0 + sj * w0_v[j, pl.ds(0, L)]
        v1 = v1 + sj * w0_v[j, pl.ds(L, L)]

    cvec = (b0_v[pl.ds(0, L)] * s0 + b0_v[pl.ds(L, L)] * s1
            + b1_v[pl.ds(0, L)] + b1_v[pl.ds(L, L)])
    cc = cvec[0]
    for l in range(1, L):
        cc = cc + cvec[l]
    vs = [v0[d] if d < L else v1[d - L] for d in range(D)]

    for cp in copies:
        cp.wait()

    riota = lax.iota(jnp.int32, L)

    def body(b, carry):
        ridx = riota + b * L
        acc = jnp.zeros((L,), jnp.float32)
        for d in range(D):
            cidx = jnp.full((L,), d, jnp.int32)
            cu = plsc.load_gather(urows_v, [ridx, cidx])
            ci = plsc.load_gather(irows_v, [ridx, cidx])
            acc = acc + cu * ci * vs[d]
        logits = acc + cc
        p = 1.0 / (1.0 + jnp.exp(-logits))
        out_v[pl.ds(b * L, L)] = p
        return carry

    lax.fori_loop(0, BPW // L, body, 0)

    pltpu.sync_copy(out_v, out_hbm.at[pl.ds(base, BPW)])


_nfm = functools.partial(
    pl.kernel,
    out_type=jax.ShapeDtypeStruct((B,), jnp.float32),
    mesh=plsc.VectorSubcoreMesh(core_axis_name="c", subcore_axis_name="s"),
    compiler_params=pltpu.CompilerParams(needs_layout_passes=False,
                                         use_tc_tiling_on_sc=False),
    scratch_types=[
        pltpu.VMEM((BPW,), jnp.int32),
        pltpu.VMEM((BPW,), jnp.int32),
        pltpu.VMEM((BPW, D), jnp.float32),
        pltpu.VMEM((BPW, D), jnp.float32),
        pltpu.VMEM((D, D), jnp.float32),
        pltpu.VMEM((D, D), jnp.float32),
        pltpu.VMEM((D,), jnp.float32),
        pltpu.VMEM((D,), jnp.float32),
        pltpu.VMEM((BPW,), jnp.float32),
        pltpu.SemaphoreType.DMA,
        pltpu.SemaphoreType.DMA,
    ],
)(_nfm_body)


@jax.jit
def kernel(user_tensor, item_tensor, user_table, item_table, W0, b0, W1, b1):
    return _nfm(user_tensor.astype(jnp.int32), item_tensor.astype(jnp.int32),
                user_table, item_table, W0, b0, W1, b1)


# speedup vs baseline: 1.5159x; 1.5159x over previous
import functools
import jax
import jax.numpy as jnp
from jax import lax
from jax.experimental import pallas as pl
from jax.experimental.pallas import tpu as pltpu
from jax.experimental.pallas import tpu_sc as plsc

B = 16384
D = 32
L = 16
NC = 2
NS = 16
NW = NC * NS
BPW = B // NW
Q = 128
NQ = BPW // Q
QB = Q // L

_GD = lax.GatherDimensionNumbers(
    offset_dims=(), collapsed_slice_dims=(0,), start_index_map=(0,))


def _perm(x, perm):
    return lax.gather(x, perm[:, None], _GD, (1,),
                      mode=lax.GatherScatterMode.PROMISE_IN_BOUNDS)


def _nfm_body(uidx_hbm, iidx_hbm, utab_hbm, itab_hbm, w_hbm, out_hbm,
              uidx_v, iidx_v, ua, ub, ia, ib, w_v, out_v,
              usemA, usemB, isemA, isemB):
    cid = lax.axis_index("c")
    sid = lax.axis_index("s")
    wid = sid * NC + cid
    base = wid * BPW

    pltpu.sync_copy(uidx_hbm.at[pl.ds(base, BPW)], uidx_v)
    pltpu.sync_copy(iidx_hbm.at[pl.ds(base, BPW)], iidx_v)

    ubufs = [ua, ub]
    ibufs = [ia, ib]
    usems = [usemA, usemB]
    isems = [isemA, isemB]

    def fire(q):
        ubuf, ibuf = ubufs[q % 2], ibufs[q % 2]
        usem, isem = usems[q % 2], isems[q % 2]

        def go(b, carry):
            u16 = uidx_v[pl.ds(q * Q + b * L, L)]
            i16 = iidx_v[pl.ds(q * Q + b * L, L)]
            for j in range(L):
                pltpu.async_copy(utab_hbm.at[pl.ds(u16[j], 1), :],
                                 ubuf.at[pl.ds(b * L + j, 1), :], usem)
                pltpu.async_copy(itab_hbm.at[pl.ds(i16[j], 1), :],
                                 ibuf.at[pl.ds(b * L + j, 1), :], isem)
            return carry

        lax.fori_loop(0, QB, go, 0)

    def drain(q):
        pltpu.make_async_copy(utab_hbm.at[pl.ds(0, Q), :],
                              ubufs[q % 2], usems[q % 2]).wait()
        pltpu.make_async_copy(itab_hbm.at[pl.ds(0, Q), :],
                              ibufs[q % 2], isems[q % 2]).wait()

    fire(0)
    fire(1)

    pltpu.sync_copy(w_hbm, w_v)

    def wrow(base_row, j, half):
        return w_v[base_row + j // 4, pl.ds((j % 4) * 32 + half * L, L)]

    s0 = jnp.zeros((L,), jnp.float32)
    s1 = jnp.zeros((L,), jnp.float32)
    for j in range(D):
        s0 = s0 + wrow(8, j, 0)
        s1 = s1 + wrow(8, j, 1)

    v0 = jnp.zeros((L,), jnp.float32)
    v1 = jnp.zeros((L,), jnp.float32)
    for j in range(D):
        sj = s0[j] if j < L else s1[j - L]
        v0 = v0 + sj * wrow(0, j, 0)
        v1 = v1 + sj * wrow(0, j, 1)

    b00 = w_v[16, pl.ds(0, L)]
    b01 = w_v[16, pl.ds(L, L)]
    b10 = w_v[16, pl.ds(2 * L, L)]
    b11 = w_v[16, pl.ds(3 * L, L)]
    cvec = b00 * s0 + b01 * s1 + b10 + b11
    cc = cvec[0]
    for l in range(1, L):
        cc = cc + cvec[l]

    lane = lax.iota(jnp.int32, L)
    perms = [lane ^ m for m in (1, 2, 4, 8)]
    masks = [lane == j for j in range(L)]

    for q in range(NQ):
        drain(q)
        ubuf, ibuf = ubufs[q % 2], ibufs[q % 2]

        def body(b, carry):
            tvec = jnp.zeros((L,), jnp.float32)
            for j in range(L):
                r = b * L + j
                u0 = ubuf[r, pl.ds(0, L)]
                u1 = ubuf[r, pl.ds(L, L)]
                i0 = ibuf[r, pl.ds(0, L)]
                i1 = ibuf[r, pl.ds(L, L)]
                y = u0 * i0 * v0 + u1 * i1 * v1
                for p in perms:
                    y = y + _perm(y, p)
                tvec = jnp.where(masks[j], y, tvec)
            pr = 1.0 / (1.0 + jnp.exp(-(tvec + cc)))
            out_v[pl.ds(q * Q + b * L, L)] = pr
            return carry

        lax.fori_loop(0, QB, body, 0)
        if q + 2 < NQ:
            fire(q + 2)

    pltpu.sync_copy(out_v, out_hbm.at[pl.ds(base, BPW)])


_nfm = functools.partial(
    pl.kernel,
    out_type=jax.ShapeDtypeStruct((B,), jnp.float32),
    mesh=plsc.VectorSubcoreMesh(core_axis_name="c", subcore_axis_name="s"),
    scratch_types=[
        pltpu.VMEM((BPW,), jnp.int32),
        pltpu.VMEM((BPW,), jnp.int32),
        pltpu.VMEM((Q, D), jnp.float32),
        pltpu.VMEM((Q, D), jnp.float32),
        pltpu.VMEM((Q, D), jnp.float32),
        pltpu.VMEM((Q, D), jnp.float32),
        pltpu.VMEM((24, 128), jnp.float32),
        pltpu.VMEM((BPW,), jnp.float32),
        pltpu.SemaphoreType.DMA,
        pltpu.SemaphoreType.DMA,
        pltpu.SemaphoreType.DMA,
        pltpu.SemaphoreType.DMA,
    ],
)(_nfm_body)


@jax.jit
def kernel(user_tensor, item_tensor, user_table, item_table, W0, b0, W1, b1):
    w_pack = jnp.concatenate([
        W0.reshape(8, 128),
        W1.reshape(8, 128),
        jnp.concatenate([b0, b1, jnp.zeros((64,), jnp.float32)]).reshape(1, 128),
        jnp.zeros((7, 128), jnp.float32),
    ], axis=0)
    return _nfm(user_tensor.astype(jnp.int32), item_tensor.astype(jnp.int32),
                user_table, item_table, w_pack)
